# baseline (speedup 1.0000x reference)
import functools

import jax
import jax.numpy as jnp
from jax import lax
from jax.experimental import pallas as pl
from jax.experimental.pallas import tpu as pltpu
from jax.experimental.pallas import tpu_sc as plsc

NC = 2
NS = 16
NW = NC * NS
LANES = 16


@functools.partial(jax.jit, static_argnums=(3, 4, 5))
def _sc_embed(ids, table2, pos2, B, L, E):
    steps_per_w = B // (2 * NW)
    scale = 8.0
    E2 = 2 * E
    W = 2 * L

    mesh = plsc.VectorSubcoreMesh(core_axis_name="c", subcore_axis_name="s")

    @functools.partial(
        pl.kernel,
        out_type=jax.ShapeDtypeStruct((B * L // 2, E2), jnp.float32),
        mesh=mesh,
        scratch_types=[
            pltpu.VMEM((W,), jnp.int32),
            pltpu.VMEM((W,), jnp.int32),
            pltpu.VMEM((W, E2), jnp.float32),
            pltpu.VMEM((W // 2, E2), jnp.float32),
            pltpu.VMEM((L // 2, E2), jnp.float32),
            pltpu.SemaphoreType.DMA,
        ],
    )
    def k(table_hbm, ids_hbm, pos_hbm, out_hbm, ids_v, idx2_v, rows_v, out_v, pos_v, sem):
        wid = lax.axis_index("s") * NC + lax.axis_index("c")

        pltpu.sync_copy(pos_hbm, pos_v)

        n_full = L // LANES
        tail_lo = LANES - L % LANES if L % LANES else LANES

        @pl.loop(0, steps_per_w)
        def _(s):
            step = wid * steps_per_w + s
            base = step * W
            pltpu.sync_copy(ids_hbm.at[pl.ds(base, W)], ids_v)

            @pl.loop(0, W // LANES)
            def _(g):
                b16 = g * LANES
                idx2_v[pl.ds(b16, LANES)] = ids_v[pl.ds(b16, LANES)] >> 1

            copies = []
            for o in range(0, W, 128):
                n = min(128, W - o)
                copies.append(
                    pltpu.async_copy(
                        table_hbm.at[idx2_v.at[pl.ds(o, n)]],
                        rows_v.at[pl.ds(o, n)],
                        sem,
                    )
                )
            for c in copies:
                c.wait()

            def do_rows(half, b16, j_lo):
                idvec = ids_v[pl.ds(half * L + b16, LANES)]
                mvec = jnp.where(idvec == 0, 0.0, 1.0)
                hvec = (idvec & 1) << 6
                for j in range(j_lo, LANES):
                    m = mvec[j]
                    h = hvec[j]
                    w = half * L + b16 + j
                    prow = (b16 >> 1) + (j >> 1)
                    orow = half * (L // 2) + prow
                    olane = (j & 1) * E
                    for c in range(E // LANES):
                        osl = pl.ds(olane + c * LANES, LANES)
                        out_v[orow, osl] = (
                            rows_v[w, pl.ds(h + c * LANES, LANES)] * scale
                            + pos_v[prow, osl]
                        ) * m

            for half in range(2):
                @pl.loop(0, n_full)
                def _(g, half=half):
                    do_rows(half, g * LANES, 0)

                if L % LANES:
                    do_rows(half, L - LANES, tail_lo)

            pltpu.sync_copy(out_v, out_hbm.at[pl.ds(step * (W // 2), W // 2)])

    return k(table2, ids, pos2)


def kernel(inputs, token_table, position_table):
    B, L = inputs.shape
    V, E = token_table.shape
    ids = inputs.reshape(-1).astype(jnp.int32)
    table2 = token_table.reshape(V // 2, 2 * E)
    pos2 = position_table.reshape(L // 2, 2 * E)
    out2 = _sc_embed(ids, table2, pos2, B, L, E)
    return out2.reshape(B, L, E)


# speedup vs baseline: 2.1319x; 2.1319x over previous
import functools

import jax
import jax.numpy as jnp
from jax import lax
from jax.experimental import pallas as pl
from jax.experimental.pallas import tpu as pltpu
from jax.experimental.pallas import tpu_sc as plsc

NC = 2
NS = 16
NW = NC * NS
LANES = 16


@functools.partial(jax.jit, static_argnums=(3, 4, 5))
def _sc_embed(ids, token_table, position_table, B, L, E):
    steps = B // NW
    n_ids = steps * L
    scale = 8.0

    mesh = plsc.VectorSubcoreMesh(core_axis_name="c", subcore_axis_name="s")

    @functools.partial(
        pl.kernel,
        out_type=jax.ShapeDtypeStruct((B * L, 2 * E), jnp.float32),
        mesh=mesh,
        scratch_types=[
            pltpu.VMEM((n_ids,), jnp.int32),
            pltpu.VMEM((L, E), jnp.float32),
            pltpu.VMEM((L, E), jnp.float32),
            pltpu.VMEM((L, E), jnp.float32),
            pltpu.VMEM((L, E), jnp.float32),
            pltpu.VMEM((L, E), jnp.float32),
            pltpu.SemaphoreType.DMA,
            pltpu.SemaphoreType.DMA,
            pltpu.SemaphoreType.DMA,
            pltpu.SemaphoreType.DMA,
        ],
        compiler_params=pltpu.CompilerParams(use_tc_tiling_on_sc=False),
    )
    def k(table_hbm, ids_hbm, pos_hbm, out_hbm, ids_all, pos_v,
          rows0, rows1, outb0, outb1, sg0, sg1, so0, so1):
        rows_v = (rows0, rows1)
        out_v = (outb0, outb1)
        sg = (sg0, sg1)
        so = (so0, so1)

        wid = lax.axis_index("s") * NC + lax.axis_index("c")
        wbase = wid * n_ids

        pltpu.sync_copy(pos_hbm, pos_v)
        pltpu.sync_copy(ids_hbm.at[pl.ds(wbase, n_ids)], ids_all)

        g_chunks = [(o, min(128, L - o)) for o in range(0, L, 128)]

        def gather_fire(b, sl):
            for o, n in g_chunks:
                pltpu.async_copy(
                    table_hbm.at[ids_all.at[pl.ds(sl * L + o, n)]],
                    rows_v[b].at[pl.ds(o, n)],
                    sg[b],
                )

        def gather_wait(b):
            for o, n in g_chunks:
                pltpu.make_async_copy(
                    table_hbm.at[ids_all.at[pl.ds(o, n)]],
                    rows_v[b].at[pl.ds(o, n)],
                    sg[b],
                ).wait()

        def out_fire(b, sl):
            pltpu.async_copy(
                out_v[b],
                out_hbm.at[pl.ds(wbase + sl * L, L), pl.ds(0, E)],
                so[b],
            )

        def out_wait(b):
            pltpu.make_async_copy(
                out_v[b],
                out_hbm.at[pl.ds(0, L), pl.ds(0, E)],
                so[b],
            ).wait()

        def compute(b, sl):
            def do_rows(b16, j_lo):
                idvec = ids_all[pl.ds(sl * L + b16, LANES)]
                mvec = jnp.where(idvec == 0, 0.0, 1.0)
                for j in range(j_lo, LANES):
                    m = mvec[j]
                    w = b16 + j
                    for c in range(E // LANES):
                        sl16 = pl.ds(c * LANES, LANES)
                        out_v[b][w, sl16] = (
                            rows_v[b][w, sl16] * scale + pos_v[w, sl16]
                        ) * m

            @pl.loop(0, L // LANES)
            def _(g):
                do_rows(g * LANES, 0)

            if L % LANES:
                do_rows(L - LANES, LANES - L % LANES)

        gather_fire(0, 0)

        @pl.loop(0, steps // 2)
        def _(ss):
            for b in range(2):
                sl = ss * 2 + b

                @pl.when(sl + 1 < steps)
                def _():
                    gather_fire(1 - b, sl + 1)

                gather_wait(b)

                @pl.when(sl >= 2)
                def _():
                    out_wait(b)

                compute(b, sl)
                out_fire(b, sl)

        out_wait(0)
        out_wait(1)

    return k(token_table, ids, position_table)


def kernel(inputs, token_table, position_table):
    B, L = inputs.shape
    V, E = token_table.shape
    ids = inputs.reshape(-1).astype(jnp.int32)
    out2 = _sc_embed(ids, token_table, position_table, B, L, E)
    return out2[:, :E].reshape(B, L, E)
